# baseline (speedup 1.0000x reference)
import math

import jax
import jax.numpy as jnp
from jax import lax
from jax.experimental import pallas as pl
from jax.experimental.pallas import tpu as pltpu
from jax.experimental.pallas import tpu_sc as plsc

N = 10000
D = 128
E = 320000
NC = 2
NS = 16
NW = NC * NS
CHUNK = 128
GC = 16
NG = 5
CH = GC * NG
EPAD = NW * CH * CHUNK
NP = 10112
RPT = NP // NS
BN = 1000


_MESH = plsc.VectorSubcoreMesh(
    core_axis_name="c", subcore_axis_name="s",
    num_cores=NC, num_subcores=NS)


def _make_sc_agg():
    scratch = [
        pltpu.VMEM((GC, CHUNK), jnp.int32),
        pltpu.VMEM((GC, CHUNK), jnp.int32),
        pltpu.VMEM((CHUNK, D), jnp.float32),
        pltpu.VMEM((CHUNK, D), jnp.float32),
        pltpu.VMEM_SHARED((NP, D), jnp.float32),
        pltpu.SemaphoreType.DMA,
        pltpu.SemaphoreType.DMA,
    ]

    def body(x_hbm, srcb, dstb, agg_out,
             src_v, dst_v, rows0, rows1, acc_sh, sem0, sem1):
        cid = lax.axis_index("c")
        sid = lax.axis_index("s")
        wid = sid * NC + cid
        base = sid * RPT
        bufs = (rows0, rows1)
        sems = (sem0, sem1)

        def zrow(i, carry):
            for j in range(D // 16):
                rows0[i, pl.ds(j * 16, 16)] = jnp.zeros((16,), jnp.float32)
            return carry
        lax.fori_loop(0, CHUNK, zrow, 0)
        for k in range(RPT // CHUNK):
            pltpu.sync_copy(rows0, acc_sh.at[pl.ds(base + k * CHUNK, CHUNK)])
        tail = RPT % CHUNK
        if tail:
            pltpu.sync_copy(
                rows0.at[pl.ds(0, tail)],
                acc_sh.at[pl.ds(base + RPT - tail, tail)])
        plsc.subcore_barrier()

        def group(g, carry):
            pltpu.sync_copy(srcb.at[wid, pl.ds(g * GC, GC)], src_v)
            pltpu.sync_copy(dstb.at[wid, pl.ds(g * GC, GC)], dst_v)
            h = pltpu.async_copy(x_hbm.at[src_v.at[0]], rows0, sem0)
            for j in range(GC):
                h.wait()
                if j + 1 < GC:
                    h = pltpu.async_copy(
                        x_hbm.at[src_v.at[j + 1]],
                        bufs[(j + 1) % 2], sems[(j + 1) % 2])
                pltpu.sync_copy(bufs[j % 2], acc_sh.at[dst_v.at[j]],
                                add=True)
            return carry
        lax.fori_loop(0, NG, group, 0)

        plsc.subcore_barrier()
        pltpu.sync_copy(acc_sh.at[pl.ds(base, RPT)],
                        agg_out.at[cid, pl.ds(base, RPT)])

    return pl.kernel(body, out_type=jax.ShapeDtypeStruct((NC, NP, D),
                                                         jnp.float32),
                     mesh=_MESH, scratch_types=scratch)


def _make_sc_cnt():
    scratch = [
        pltpu.VMEM((GC, CHUNK), jnp.int32),
        pltpu.VMEM((CHUNK, D), jnp.float32),
        pltpu.VMEM_SHARED((NP, D), jnp.float32),
    ]

    def body(dstb, cnt_out, dst_v, ones_v, cnt_sh):
        cid = lax.axis_index("c")
        sid = lax.axis_index("s")
        wid = sid * NC + cid
        base = sid * RPT

        def zcnt(i, carry):
            for j in range(D // 16):
                ones_v[i, pl.ds(j * 16, 16)] = jnp.zeros((16,), jnp.float32)
            return carry
        lax.fori_loop(0, CHUNK, zcnt, 0)
        for k in range(RPT // CHUNK):
            pltpu.sync_copy(ones_v, cnt_sh.at[pl.ds(base + k * CHUNK, CHUNK)])
        tail = RPT % CHUNK
        if tail:
            pltpu.sync_copy(
                ones_v.at[pl.ds(0, tail)],
                cnt_sh.at[pl.ds(base + RPT - tail, tail)])

        def ocnt(i, carry):
            for j in range(D // 16):
                ones_v[i, pl.ds(j * 16, 16)] = jnp.ones((16,), jnp.float32)
            return carry
        lax.fori_loop(0, CHUNK, ocnt, 0)
        plsc.subcore_barrier()

        def group(g, carry):
            pltpu.sync_copy(dstb.at[wid, pl.ds(g * GC, GC)], dst_v)
            for j in range(GC):
                pltpu.sync_copy(ones_v, cnt_sh.at[dst_v.at[j]], add=True)
            return carry
        lax.fori_loop(0, NG, group, 0)

        plsc.subcore_barrier()
        pltpu.sync_copy(cnt_sh.at[pl.ds(base, RPT)],
                        cnt_out.at[cid, pl.ds(base, RPT)])

    return pl.kernel(body, out_type=jax.ShapeDtypeStruct((NC, NP, D),
                                                         jnp.float32),
                     mesh=_MESH, scratch_types=scratch)


_sc_agg = _make_sc_agg()
_sc_cnt = _make_sc_cnt()


def _tc_layer(aggp, cntp, xin, wlT, wrT, b):
    def tc_body(agg_ref, cnt_ref, x_ref, wl_ref, wr_ref, b_ref, o_ref):
        agg = agg_ref[0] + agg_ref[1]
        cnt = cnt_ref[0, :, 0:1] + cnt_ref[1, :, 0:1]
        mean = agg / jnp.maximum(cnt, 1.0)
        xb = x_ref[...]
        h = jnp.dot(mean, wl_ref[...], preferred_element_type=jnp.float32)
        h += jnp.dot(xb, wr_ref[...], preferred_element_type=jnp.float32)
        h += b_ref[...]
        o_ref[...] = jnp.maximum(h, 0.0) + xb

    return pl.pallas_call(
        tc_body,
        grid=(N // BN,),
        in_specs=[
            pl.BlockSpec((NC, BN, D), lambda i: (0, i, 0)),
            pl.BlockSpec((NC, BN, D), lambda i: (0, i, 0)),
            pl.BlockSpec((BN, D), lambda i: (i, 0)),
            pl.BlockSpec((D, D), lambda i: (0, 0)),
            pl.BlockSpec((D, D), lambda i: (0, 0)),
            pl.BlockSpec((1, D), lambda i: (0, 0)),
        ],
        out_specs=pl.BlockSpec((BN, D), lambda i: (i, 0)),
        out_shape=jax.ShapeDtypeStruct((N, D), jnp.float32),
    )(aggp, cntp, xin, wlT, wrT, b)


def _tc_layer_head(aggp, cntp, hin, wlT, wrT, b, wo1T, bo1p, wo2T, bo2p):
    def tc_body(agg_ref, cnt_ref, h_ref, wl_ref, wr_ref, b_ref,
                wo1_ref, bo1_ref, wo2_ref, bo2_ref, o_ref):
        agg = agg_ref[0] + agg_ref[1]
        cnt = cnt_ref[0, :, 0:1] + cnt_ref[1, :, 0:1]
        mean = agg / jnp.maximum(cnt, 1.0)
        hb = h_ref[...]
        t = jnp.dot(mean, wl_ref[...], preferred_element_type=jnp.float32)
        t += jnp.dot(hb, wr_ref[...], preferred_element_type=jnp.float32)
        t += b_ref[...]
        h2 = jnp.maximum(t, 0.0) + hb
        h3 = jnp.dot(h2, wo1_ref[...], preferred_element_type=jnp.float32)
        h3 = jnp.maximum(h3 + bo1_ref[...], 0.0)
        o = jnp.dot(h3, wo2_ref[...], preferred_element_type=jnp.float32)
        o_ref[...] = o + bo2_ref[...]

    full = lambda i: (0, 0)
    return pl.pallas_call(
        tc_body,
        grid=(N // BN,),
        in_specs=[
            pl.BlockSpec((NC, BN, D), lambda i: (0, i, 0)),
            pl.BlockSpec((NC, BN, D), lambda i: (0, i, 0)),
            pl.BlockSpec((BN, D), lambda i: (i, 0)),
            pl.BlockSpec((D, D), full),
            pl.BlockSpec((D, D), full),
            pl.BlockSpec((1, D), full),
            pl.BlockSpec((D, D), full),
            pl.BlockSpec((1, D), full),
            pl.BlockSpec((D, D), full),
            pl.BlockSpec((1, D), full),
        ],
        out_specs=pl.BlockSpec((BN, D), lambda i: (i, 0)),
        out_shape=jax.ShapeDtypeStruct((N, D), jnp.float32),
    )(aggp, cntp, hin, wlT, wrT, b, wo1T, bo1p, wo2T, bo2p)


def kernel(x, edge_index, Wl0, bl0, Wr0, g0, be0, Wl1, bl1, Wr1, g1, be1,
           Wo1, bo1, Wo2, bo2):
    src = edge_index[0]
    dst = edge_index[1]
    pad = EPAD - E
    ar = jnp.arange(pad, dtype=jnp.int32)
    psrc = ar % N
    pdst = N + (ar % 16)
    srcb = jnp.concatenate([src, psrc]).reshape(NW, CH, CHUNK)
    dstb = jnp.concatenate([dst, pdst]).reshape(NW, CH, CHUNK)

    inv = 1.0 / math.sqrt(1.0 + 1e-5)
    s0 = g0 * inv
    s1 = g1 * inv
    wl0T = (Wl0 * s0[:, None]).T
    wr0T = (Wr0 * s0[:, None]).T
    b0 = (bl0 * s0 + be0).reshape(1, D)
    wl1T = (Wl1 * s1[:, None]).T
    wr1T = (Wr1 * s1[:, None]).T
    b1 = (bl1 * s1 + be1).reshape(1, D)
    wo1T = jnp.pad(Wo1.T, ((0, 0), (0, D - Wo1.shape[0])))
    bo1p = jnp.pad(bo1, (0, D - bo1.shape[0])).reshape(1, D)
    wo2T = jnp.pad(Wo2.T, ((0, D - Wo2.shape[1]), (0, D - Wo2.shape[0])))
    bo2p = jnp.pad(bo2, (0, D - bo2.shape[0])).reshape(1, D)

    cnt0p = _sc_cnt(dstb)
    dep = jnp.where(cnt0p[0, 0, 0] < 0.0, 1, 0).astype(jnp.int32)
    agg0p = _sc_agg(x, srcb + dep, dstb)
    h = _tc_layer(agg0p, cnt0p, x, wl0T, wr0T, b0)

    agg1p = _sc_agg(h, srcb, dstb)
    out = _tc_layer_head(agg1p, cnt0p, h, wl1T, wr1T, b1,
                         wo1T, bo1p, wo2T, bo2p)
    return out[:, :Wo2.shape[0]]


# speedup vs baseline: 1.2724x; 1.2724x over previous
import math

import jax
import jax.numpy as jnp
from jax import lax
from jax.experimental import pallas as pl
from jax.experimental.pallas import tpu as pltpu
from jax.experimental.pallas import tpu_sc as plsc

N = 10000
D = 128
E = 320000
NC = 2
NS = 16
NW = NC * NS
CHUNK = 128
GC = 16
NG = 5
CH = GC * NG
EPAD = NW * CH * CHUNK
NP = 10112
RPT = NP // NS
BN = 1000
CW = 16


_MESH = plsc.VectorSubcoreMesh(
    core_axis_name="c", subcore_axis_name="s",
    num_cores=NC, num_subcores=NS)


def _make_sc_agg():
    scratch = [
        pltpu.VMEM((GC, CHUNK), jnp.int32),
        pltpu.VMEM((GC, CHUNK), jnp.int32),
        pltpu.VMEM((CHUNK, D), jnp.float32),
        pltpu.VMEM((CHUNK, D), jnp.float32),
        pltpu.VMEM_SHARED((NP, D), jnp.float32),
        pltpu.SemaphoreType.DMA,
        pltpu.SemaphoreType.DMA,
    ]

    def body(x_hbm, srcb, dstb, agg_out,
             src_v, dst_v, rows0, rows1, acc_sh, sem0, sem1):
        cid = lax.axis_index("c")
        sid = lax.axis_index("s")
        wid = sid * NC + cid
        base = sid * RPT
        bufs = (rows0, rows1)
        sems = (sem0, sem1)

        def zrow(i, carry):
            for j in range(D // 16):
                rows0[i, pl.ds(j * 16, 16)] = jnp.zeros((16,), jnp.float32)
            return carry
        lax.fori_loop(0, CHUNK, zrow, 0)
        for k in range(RPT // CHUNK):
            pltpu.sync_copy(rows0, acc_sh.at[pl.ds(base + k * CHUNK, CHUNK)])
        tail = RPT % CHUNK
        if tail:
            pltpu.sync_copy(
                rows0.at[pl.ds(0, tail)],
                acc_sh.at[pl.ds(base + RPT - tail, tail)])
        plsc.subcore_barrier()

        def group(g, carry):
            pltpu.sync_copy(srcb.at[wid, pl.ds(g * GC, GC)], src_v)
            pltpu.sync_copy(dstb.at[wid, pl.ds(g * GC, GC)], dst_v)
            handles = [pltpu.async_copy(x_hbm.at[src_v.at[0]], rows0, sem0)]
            for j in range(GC):
                if j + 1 < GC:
                    handles.append(pltpu.async_copy(
                        x_hbm.at[src_v.at[j + 1]],
                        bufs[(j + 1) % 2], sems[(j + 1) % 2]))
                handles[j].wait()
                pltpu.sync_copy(bufs[j % 2], acc_sh.at[dst_v.at[j]],
                                add=True)
            return carry
        lax.fori_loop(0, NG, group, 0)

        plsc.subcore_barrier()
        pltpu.sync_copy(acc_sh.at[pl.ds(base, RPT)],
                        agg_out.at[cid, pl.ds(base, RPT)])

    return pl.kernel(body, out_type=jax.ShapeDtypeStruct((NC, NP, D),
                                                         jnp.float32),
                     mesh=_MESH, scratch_types=scratch)


def _make_sc_cnt():
    scratch = [
        pltpu.VMEM((GC, CHUNK), jnp.int32),
        pltpu.VMEM((CHUNK, CW), jnp.float32),
        pltpu.VMEM_SHARED((NP, CW), jnp.float32),
    ]

    def body(dstb, cnt_out, dst_v, ones_v, cnt_sh):
        cid = lax.axis_index("c")
        sid = lax.axis_index("s")
        wid = sid * NC + cid
        base = sid * RPT

        def zcnt(i, carry):
            ones_v[i] = jnp.zeros((CW,), jnp.float32)
            return carry
        lax.fori_loop(0, CHUNK, zcnt, 0)
        for k in range(RPT // CHUNK):
            pltpu.sync_copy(ones_v, cnt_sh.at[pl.ds(base + k * CHUNK, CHUNK)])
        tail = RPT % CHUNK
        if tail:
            pltpu.sync_copy(
                ones_v.at[pl.ds(0, tail)],
                cnt_sh.at[pl.ds(base + RPT - tail, tail)])

        def ocnt(i, carry):
            ones_v[i] = jnp.ones((CW,), jnp.float32)
            return carry
        lax.fori_loop(0, CHUNK, ocnt, 0)
        plsc.subcore_barrier()

        def group(g, carry):
            pltpu.sync_copy(dstb.at[wid, pl.ds(g * GC, GC)], dst_v)
            for j in range(GC):
                pltpu.sync_copy(ones_v, cnt_sh.at[dst_v.at[j]], add=True)
            return carry
        lax.fori_loop(0, NG, group, 0)

        plsc.subcore_barrier()
        pltpu.sync_copy(cnt_sh.at[pl.ds(base, RPT)],
                        cnt_out.at[cid, pl.ds(base, RPT)])

    return pl.kernel(
        body, out_type=jax.ShapeDtypeStruct((NC, NP, CW), jnp.float32),
        mesh=_MESH, scratch_types=scratch,
        compiler_params=pltpu.CompilerParams(use_tc_tiling_on_sc=False))


_sc_agg = _make_sc_agg()
_sc_cnt = _make_sc_cnt()


def _tc_layer(aggp, cntp, xin, wlT, wrT, b):
    def tc_body(agg_ref, cnt_ref, x_ref, wl_ref, wr_ref, b_ref, o_ref):
        agg = agg_ref[0] + agg_ref[1]
        cnt = (cnt_ref[0] + cnt_ref[1])[:, 0:1]
        mean = agg / jnp.maximum(cnt, 1.0)
        xb = x_ref[...]
        h = jnp.dot(mean, wl_ref[...], preferred_element_type=jnp.float32)
        h += jnp.dot(xb, wr_ref[...], preferred_element_type=jnp.float32)
        h += b_ref[...]
        o_ref[...] = jnp.maximum(h, 0.0) + xb

    return pl.pallas_call(
        tc_body,
        grid=(N // BN,),
        in_specs=[
            pl.BlockSpec((NC, BN, D), lambda i: (0, i, 0)),
            pl.BlockSpec((NC, BN, CW), lambda i: (0, i, 0)),
            pl.BlockSpec((BN, D), lambda i: (i, 0)),
            pl.BlockSpec((D, D), lambda i: (0, 0)),
            pl.BlockSpec((D, D), lambda i: (0, 0)),
            pl.BlockSpec((1, D), lambda i: (0, 0)),
        ],
        out_specs=pl.BlockSpec((BN, D), lambda i: (i, 0)),
        out_shape=jax.ShapeDtypeStruct((N, D), jnp.float32),
    )(aggp, cntp, xin, wlT, wrT, b)


def _tc_layer_head(aggp, cntp, hin, wlT, wrT, b, wo1T, bo1p, wo2T, bo2p):
    def tc_body(agg_ref, cnt_ref, h_ref, wl_ref, wr_ref, b_ref,
                wo1_ref, bo1_ref, wo2_ref, bo2_ref, o_ref):
        agg = agg_ref[0] + agg_ref[1]
        cnt = (cnt_ref[0] + cnt_ref[1])[:, 0:1]
        mean = agg / jnp.maximum(cnt, 1.0)
        hb = h_ref[...]
        t = jnp.dot(mean, wl_ref[...], preferred_element_type=jnp.float32)
        t += jnp.dot(hb, wr_ref[...], preferred_element_type=jnp.float32)
        t += b_ref[...]
        h2 = jnp.maximum(t, 0.0) + hb
        h3 = jnp.dot(h2, wo1_ref[...], preferred_element_type=jnp.float32)
        h3 = jnp.maximum(h3 + bo1_ref[...], 0.0)
        o = jnp.dot(h3, wo2_ref[...], preferred_element_type=jnp.float32)
        o_ref[...] = o + bo2_ref[...]

    full = lambda i: (0, 0)
    return pl.pallas_call(
        tc_body,
        grid=(N // BN,),
        in_specs=[
            pl.BlockSpec((NC, BN, D), lambda i: (0, i, 0)),
            pl.BlockSpec((NC, BN, CW), lambda i: (0, i, 0)),
            pl.BlockSpec((BN, D), lambda i: (i, 0)),
            pl.BlockSpec((D, D), full),
            pl.BlockSpec((D, D), full),
            pl.BlockSpec((1, D), full),
            pl.BlockSpec((D, D), full),
            pl.BlockSpec((1, D), full),
            pl.BlockSpec((D, D), full),
            pl.BlockSpec((1, D), full),
        ],
        out_specs=pl.BlockSpec((BN, D), lambda i: (i, 0)),
        out_shape=jax.ShapeDtypeStruct((N, D), jnp.float32),
    )(aggp, cntp, hin, wlT, wrT, b, wo1T, bo1p, wo2T, bo2p)


def kernel(x, edge_index, Wl0, bl0, Wr0, g0, be0, Wl1, bl1, Wr1, g1, be1,
           Wo1, bo1, Wo2, bo2):
    src = edge_index[0]
    dst = edge_index[1]
    pad = EPAD - E
    ar = jnp.arange(pad, dtype=jnp.int32)
    psrc = ar % N
    pdst = N + (ar % 16)
    srcb = jnp.concatenate([src, psrc]).reshape(NW, CH, CHUNK)
    dstb = jnp.concatenate([dst, pdst]).reshape(NW, CH, CHUNK)

    inv = 1.0 / math.sqrt(1.0 + 1e-5)
    s0 = g0 * inv
    s1 = g1 * inv
    wl0T = (Wl0 * s0[:, None]).T
    wr0T = (Wr0 * s0[:, None]).T
    b0 = (bl0 * s0 + be0).reshape(1, D)
    wl1T = (Wl1 * s1[:, None]).T
    wr1T = (Wr1 * s1[:, None]).T
    b1 = (bl1 * s1 + be1).reshape(1, D)
    wo1T = jnp.pad(Wo1.T, ((0, 0), (0, D - Wo1.shape[0])))
    bo1p = jnp.pad(bo1, (0, D - bo1.shape[0])).reshape(1, D)
    wo2T = jnp.pad(Wo2.T, ((0, D - Wo2.shape[1]), (0, D - Wo2.shape[0])))
    bo2p = jnp.pad(bo2, (0, D - bo2.shape[0])).reshape(1, D)

    cnt0p = _sc_cnt(dstb)
    agg0p = _sc_agg(x, srcb, dstb)
    h = _tc_layer(agg0p, cnt0p, x, wl0T, wr0T, b0)

    agg1p = _sc_agg(h, srcb, dstb)
    out = _tc_layer_head(agg1p, cnt0p, h, wl1T, wr1T, b1,
                         wo1T, bo1p, wo2T, bo2p)
    return out[:, :Wo2.shape[0]]


# speedup vs baseline: 1.2980x; 1.0201x over previous
import math

import jax
import jax.numpy as jnp
from jax import lax
from jax.experimental import pallas as pl
from jax.experimental.pallas import tpu as pltpu
from jax.experimental.pallas import tpu_sc as plsc

N = 10000
D = 128
E = 320000
NC = 2
NS = 16
NW = NC * NS
CHUNK = 128
GC = 16
NG = 5
CH = GC * NG
EPAD = NW * CH * CHUNK
NP = 10112
RPT = NP // NS
BN = 2000
CW = 16


_MESH = plsc.VectorSubcoreMesh(
    core_axis_name="c", subcore_axis_name="s",
    num_cores=NC, num_subcores=NS)


def _make_sc_agg():
    scratch = [
        pltpu.VMEM((GC, CHUNK), jnp.int32),
        pltpu.VMEM((GC, CHUNK), jnp.int32),
        pltpu.VMEM((CHUNK, D), jnp.float32),
        pltpu.VMEM((CHUNK, D), jnp.float32),
        pltpu.VMEM_SHARED((NP, D), jnp.float32),
        pltpu.SemaphoreType.DMA,
        pltpu.SemaphoreType.DMA,
    ]

    def body(x_hbm, srcb, dstb, agg_out,
             src_v, dst_v, rows0, rows1, acc_sh, sem0, sem1):
        cid = lax.axis_index("c")
        sid = lax.axis_index("s")
        wid = sid * NC + cid
        base = sid * RPT
        bufs = (rows0, rows1)
        sems = (sem0, sem1)

        def zrow(i, carry):
            for j in range(D // 16):
                rows0[i, pl.ds(j * 16, 16)] = jnp.zeros((16,), jnp.float32)
            return carry
        lax.fori_loop(0, CHUNK, zrow, 0)
        for k in range(RPT // CHUNK):
            pltpu.sync_copy(rows0, acc_sh.at[pl.ds(base + k * CHUNK, CHUNK)])
        tail = RPT % CHUNK
        if tail:
            pltpu.sync_copy(
                rows0.at[pl.ds(0, tail)],
                acc_sh.at[pl.ds(base + RPT - tail, tail)])
        plsc.subcore_barrier()

        def group(g, carry):
            pltpu.sync_copy(srcb.at[wid, pl.ds(g * GC, GC)], src_v)
            pltpu.sync_copy(dstb.at[wid, pl.ds(g * GC, GC)], dst_v)
            handles = [pltpu.async_copy(x_hbm.at[src_v.at[0]], rows0, sem0)]
            for j in range(GC):
                if j + 1 < GC:
                    handles.append(pltpu.async_copy(
                        x_hbm.at[src_v.at[j + 1]],
                        bufs[(j + 1) % 2], sems[(j + 1) % 2]))
                handles[j].wait()
                pltpu.sync_copy(bufs[j % 2], acc_sh.at[dst_v.at[j]],
                                add=True)
            return carry
        lax.fori_loop(0, NG, group, 0)

        plsc.subcore_barrier()
        pltpu.sync_copy(acc_sh.at[pl.ds(base, RPT)],
                        agg_out.at[cid, pl.ds(base, RPT)])

    return pl.kernel(body, out_type=jax.ShapeDtypeStruct((NC, NP, D),
                                                         jnp.float32),
                     mesh=_MESH, scratch_types=scratch)


def _make_sc_cnt():
    scratch = [
        pltpu.VMEM((GC, CHUNK), jnp.int32),
        pltpu.VMEM((CHUNK, CW), jnp.float32),
        pltpu.VMEM_SHARED((NP, CW), jnp.float32),
    ]

    def body(dstb, cnt_out, dst_v, ones_v, cnt_sh):
        cid = lax.axis_index("c")
        sid = lax.axis_index("s")
        wid = sid * NC + cid
        base = sid * RPT

        def zcnt(i, carry):
            ones_v[i] = jnp.zeros((CW,), jnp.float32)
            return carry
        lax.fori_loop(0, CHUNK, zcnt, 0)
        for k in range(RPT // CHUNK):
            pltpu.sync_copy(ones_v, cnt_sh.at[pl.ds(base + k * CHUNK, CHUNK)])
        tail = RPT % CHUNK
        if tail:
            pltpu.sync_copy(
                ones_v.at[pl.ds(0, tail)],
                cnt_sh.at[pl.ds(base + RPT - tail, tail)])

        def ocnt(i, carry):
            ones_v[i] = jnp.ones((CW,), jnp.float32)
            return carry
        lax.fori_loop(0, CHUNK, ocnt, 0)
        plsc.subcore_barrier()

        def group(g, carry):
            pltpu.sync_copy(dstb.at[wid, pl.ds(g * GC, GC)], dst_v)
            for j in range(GC):
                pltpu.sync_copy(ones_v, cnt_sh.at[dst_v.at[j]], add=True)
            return carry
        lax.fori_loop(0, NG, group, 0)

        plsc.subcore_barrier()
        pltpu.sync_copy(cnt_sh.at[pl.ds(base, RPT)],
                        cnt_out.at[cid, pl.ds(base, RPT)])

    return pl.kernel(
        body, out_type=jax.ShapeDtypeStruct((NC, NP, CW), jnp.float32),
        mesh=_MESH, scratch_types=scratch,
        compiler_params=pltpu.CompilerParams(use_tc_tiling_on_sc=False))


_sc_agg = _make_sc_agg()
_sc_cnt = _make_sc_cnt()


def _tc_layer(aggp, cntp, xin, wlT, wrT, b):
    def tc_body(agg_ref, cnt_ref, x_ref, wl_ref, wr_ref, b_ref, o_ref):
        agg = agg_ref[0] + agg_ref[1]
        cnt = (cnt_ref[0] + cnt_ref[1])[:, 0:1]
        mean = agg / jnp.maximum(cnt, 1.0)
        xb = x_ref[...]
        h = jnp.dot(mean, wl_ref[...], preferred_element_type=jnp.float32)
        h += jnp.dot(xb, wr_ref[...], preferred_element_type=jnp.float32)
        h += b_ref[...]
        o_ref[...] = jnp.maximum(h, 0.0) + xb

    return pl.pallas_call(
        tc_body,
        grid=(N // BN,),
        in_specs=[
            pl.BlockSpec((NC, BN, D), lambda i: (0, i, 0)),
            pl.BlockSpec((NC, BN, CW), lambda i: (0, i, 0)),
            pl.BlockSpec((BN, D), lambda i: (i, 0)),
            pl.BlockSpec((D, D), lambda i: (0, 0)),
            pl.BlockSpec((D, D), lambda i: (0, 0)),
            pl.BlockSpec((1, D), lambda i: (0, 0)),
        ],
        out_specs=pl.BlockSpec((BN, D), lambda i: (i, 0)),
        out_shape=jax.ShapeDtypeStruct((N, D), jnp.float32),
    )(aggp, cntp, xin, wlT, wrT, b)


def _tc_layer_head(aggp, cntp, hin, wlT, wrT, b, wo1T, bo1p, wo2T, bo2p):
    def tc_body(agg_ref, cnt_ref, h_ref, wl_ref, wr_ref, b_ref,
                wo1_ref, bo1_ref, wo2_ref, bo2_ref, o_ref):
        agg = agg_ref[0] + agg_ref[1]
        cnt = (cnt_ref[0] + cnt_ref[1])[:, 0:1]
        mean = agg / jnp.maximum(cnt, 1.0)
        hb = h_ref[...]
        t = jnp.dot(mean, wl_ref[...], preferred_element_type=jnp.float32)
        t += jnp.dot(hb, wr_ref[...], preferred_element_type=jnp.float32)
        t += b_ref[...]
        h2 = jnp.maximum(t, 0.0) + hb
        h3 = jnp.dot(h2, wo1_ref[...], preferred_element_type=jnp.float32)
        h3 = jnp.maximum(h3 + bo1_ref[...], 0.0)
        o = jnp.dot(h3, wo2_ref[...], preferred_element_type=jnp.float32)
        o_ref[...] = o + bo2_ref[...]

    full = lambda i: (0, 0)
    return pl.pallas_call(
        tc_body,
        grid=(N // BN,),
        in_specs=[
            pl.BlockSpec((NC, BN, D), lambda i: (0, i, 0)),
            pl.BlockSpec((NC, BN, CW), lambda i: (0, i, 0)),
            pl.BlockSpec((BN, D), lambda i: (i, 0)),
            pl.BlockSpec((D, D), full),
            pl.BlockSpec((D, D), full),
            pl.BlockSpec((1, D), full),
            pl.BlockSpec((D, D), full),
            pl.BlockSpec((1, D), full),
            pl.BlockSpec((D, D), full),
            pl.BlockSpec((1, D), full),
        ],
        out_specs=pl.BlockSpec((BN, D), lambda i: (i, 0)),
        out_shape=jax.ShapeDtypeStruct((N, D), jnp.float32),
    )(aggp, cntp, hin, wlT, wrT, b, wo1T, bo1p, wo2T, bo2p)


def kernel(x, edge_index, Wl0, bl0, Wr0, g0, be0, Wl1, bl1, Wr1, g1, be1,
           Wo1, bo1, Wo2, bo2):
    src = edge_index[0]
    dst = edge_index[1]
    pad = EPAD - E
    ar = jnp.arange(pad, dtype=jnp.int32)
    psrc = ar % N
    pdst = N + (ar % 16)
    srcb = jnp.concatenate([src, psrc]).reshape(NW, CH, CHUNK)
    dstb = jnp.concatenate([dst, pdst]).reshape(NW, CH, CHUNK)

    inv = 1.0 / math.sqrt(1.0 + 1e-5)
    s0 = g0 * inv
    s1 = g1 * inv
    wl0T = (Wl0 * s0[:, None]).T
    wr0T = (Wr0 * s0[:, None]).T
    b0 = (bl0 * s0 + be0).reshape(1, D)
    wl1T = (Wl1 * s1[:, None]).T
    wr1T = (Wr1 * s1[:, None]).T
    b1 = (bl1 * s1 + be1).reshape(1, D)
    wo1T = jnp.pad(Wo1.T, ((0, 0), (0, D - Wo1.shape[0])))
    bo1p = jnp.pad(bo1, (0, D - bo1.shape[0])).reshape(1, D)
    wo2T = jnp.pad(Wo2.T, ((0, D - Wo2.shape[1]), (0, D - Wo2.shape[0])))
    bo2p = jnp.pad(bo2, (0, D - bo2.shape[0])).reshape(1, D)

    cnt0p = _sc_cnt(dstb)
    agg0p = _sc_agg(x, srcb, dstb)
    h = _tc_layer(agg0p, cnt0p, x, wl0T, wr0T, b0)

    agg1p = _sc_agg(h, srcb, dstb)
    out = _tc_layer_head(agg1p, cnt0p, h, wl1T, wr1T, b1,
                         wo1T, bo1p, wo2T, bo2p)
    return out[:, :Wo2.shape[0]]
